# baseline (speedup 1.0000x reference)
import jax
import jax.numpy as jnp
from jax.experimental import pallas as pl
from jax.experimental.pallas import tpu as pltpu


def _loss_kernel(msk_ref, img_ref, lbl_ref, out_ref, acc_ref):
    i = pl.program_id(0)
    j = pl.program_id(1)

    @pl.when((i == 0) & (j == 0))
    def _init():
        acc_ref[0] = 0.0
        acc_ref[1] = 0.0

    d = img_ref[...] - lbl_ref[...]
    d2s = jnp.sum(d * d, axis=1)
    mf = msk_ref[...].astype(jnp.float32)
    acc_ref[0] += jnp.sum(d2s * mf)
    acc_ref[1] += jnp.sum(mf)

    @pl.when((i == pl.num_programs(0) - 1) & (j == pl.num_programs(1) - 1))
    def _fin():
        out_ref[0] = acc_ref[0] / (3.0 * acc_ref[1])


_H_SPLIT = 2


def kernel(image, label, mask_location):
    B, C, H, W = image.shape
    hb = H // _H_SPLIT
    out = pl.pallas_call(
        _loss_kernel,
        grid=(B, _H_SPLIT),
        in_specs=[
            pl.BlockSpec((1, hb, W), lambda i, j: (i, j, 0)),
            pl.BlockSpec((1, C, hb, W), lambda i, j: (i, 0, j, 0)),
            pl.BlockSpec((1, C, hb, W), lambda i, j: (i, 0, j, 0)),
        ],
        out_specs=pl.BlockSpec(memory_space=pltpu.SMEM),
        out_shape=jax.ShapeDtypeStruct((1,), jnp.float32),
        scratch_shapes=[pltpu.SMEM((2,), jnp.float32)],
    )(mask_location, image, label)
    return out[0]


# speedup vs baseline: 1.1158x; 1.1158x over previous
import jax
import jax.numpy as jnp
from jax.experimental import pallas as pl
from jax.experimental.pallas import tpu as pltpu


def _loss_kernel(msk_ref, img_ref, lbl_ref, out_ref, acc_ref):
    i = pl.program_id(0)
    j = pl.program_id(1)

    @pl.when((i == 0) & (j == 0))
    def _init():
        acc_ref[0] = 0.0
        acc_ref[1] = 0.0

    d = img_ref[...] - lbl_ref[...]
    d2s = jnp.sum(d * d, axis=1)
    mf = msk_ref[...].astype(jnp.float32)
    acc_ref[0] += jnp.sum(d2s * mf)
    acc_ref[1] += jnp.sum(mf)

    @pl.when((i == pl.num_programs(0) - 1) & (j == pl.num_programs(1) - 1))
    def _fin():
        out_ref[0] = acc_ref[0] / (3.0 * acc_ref[1])


_H_SPLIT = 1
_NB = 2


def kernel(image, label, mask_location):
    B, C, H, W = image.shape
    hb = H // _H_SPLIT
    out = pl.pallas_call(
        _loss_kernel,
        grid=(B // _NB, _H_SPLIT),
        in_specs=[
            pl.BlockSpec((_NB, hb, W), lambda i, j: (i, j, 0)),
            pl.BlockSpec((_NB, C, hb, W), lambda i, j: (i, 0, j, 0)),
            pl.BlockSpec((_NB, C, hb, W), lambda i, j: (i, 0, j, 0)),
        ],
        out_specs=pl.BlockSpec(memory_space=pltpu.SMEM),
        out_shape=jax.ShapeDtypeStruct((1,), jnp.float32),
        scratch_shapes=[pltpu.SMEM((2,), jnp.float32)],
    )(mask_location, image, label)
    return out[0]
